# baseline (speedup 1.0000x reference)
import functools

import jax
import jax.numpy as jnp
from jax import lax
from jax.experimental import pallas as pl
from jax.experimental.pallas import tpu as pltpu
from jax.experimental.pallas import tpu_sc as plsc

N = 10000
D = 128
E = 100000
R = 3

NC = 2
NS = 16
NW = NC * NS

CH = 128
K0 = 50
K1 = 0
EPAD = NS * (K0 + K1) * CH

NPAD = 10112
RPT = NPAD // NS
ZROWS = RPT // 8
NZCOPY = RPT // ZROWS

DPAD = 10240
DPT = DPAD // NS


def _sc_agg_body(with_deg, z_ref, s0, d0, s1, d1, s2, d2, *rest):
    if with_deg:
        (o0, o1, o2, g0, g1, g2, zb, zb1, ones, sv0, sv1, dv0, dv1, rows0,
         rows1, acc, dacc, sem0, sem1) = rest
        gouts = (g0, g1, g2)
    else:
        (o0, o1, o2, zb, zb1, ones, sv0, sv1, dv0, dv1, rows0, rows1, acc,
         dacc, sem0, sem1) = rest
        gouts = (None, None, None)
    svs = (sv0, sv1)
    dvs = (dv0, dv1)
    rowss = (rows0, rows1)
    sems = (sem0, sem1)
    srcs = (s0, s1, s2)
    dsts = (d0, d1, d2)
    outs = (o0, o1, o2)

    c = lax.axis_index("c")
    s = lax.axis_index("s")
    w = c * NS + s

    def fill_zb(i, _):
        for k in range(D // 16):
            zb[i, pl.ds(k * 16, 16)] = jnp.zeros((16,), jnp.float32)
        return _

    lax.fori_loop(jnp.int32(0), jnp.int32(ZROWS), fill_zb, jnp.int32(0))

    def fill_zb1(i, _):
        zb1[pl.ds(i * 16, 16)] = jnp.zeros((16,), jnp.float32)
        return _

    lax.fori_loop(jnp.int32(0), jnp.int32(DPT // 16), fill_zb1, jnp.int32(0))

    if with_deg:
        def fill_ones(i, _):
            ones[pl.ds(i * 16, 16)] = jnp.ones((16,), jnp.float32)
            return _

        lax.fori_loop(jnp.int32(0), jnp.int32(CH // 16), fill_ones, jnp.int32(0))

    rbase = s * RPT
    dbase = s * DPT
    ebase = jnp.where(c == 0, s * (K0 * CH), NS * (K0 * CH) + s * (K1 * CH))
    kcount = jnp.where(c == 0, jnp.int32(K0), jnp.int32(K1))
    khalf = jnp.where(c == 0, jnp.int32(K0 // 2), jnp.int32(K1 // 2))

    for r in range(R):
        with jax.named_scope("zero%d" % r):
            for z in range(NZCOPY):
                pltpu.sync_copy(zb, acc.at[pl.ds(rbase + z * ZROWS, ZROWS)])
            if with_deg:
                pltpu.sync_copy(zb1, dacc.at[pl.ds(dbase, DPT)])
            plsc.subcore_barrier()

        def prefetch(i, p):
            off = ebase + i * CH
            pltpu.sync_copy(srcs[r].at[pl.ds(off, CH)], svs[p])
            pltpu.sync_copy(dsts[r].at[pl.ds(off, CH)], dvs[p])
            pltpu.async_copy(z_ref.at[svs[p]], rowss[p], sems[p])

        def consume(p):
            pltpu.make_async_copy(z_ref.at[svs[p]], rowss[p], sems[p]).wait()
            pltpu.sync_copy(rowss[p], acc.at[dvs[p]], add=True)
            if with_deg:
                pltpu.sync_copy(ones, dacc.at[dvs[p]], add=True)

        def pair(j, carry):
            consume(0)

            @pl.when(2 * j + 2 < kcount)
            def _():
                prefetch(2 * j + 2, 0)

            consume(1)

            @pl.when(2 * j + 3 < kcount)
            def _():
                prefetch(2 * j + 3, 1)

            return carry

        with jax.named_scope("chunks%d" % r):
            @pl.when(jnp.int32(0) < kcount)
            def _():
                prefetch(jnp.int32(0), 0)
                prefetch(jnp.int32(1), 1)

            lax.fori_loop(jnp.int32(0), khalf, pair, jnp.int32(0))
            plsc.subcore_barrier()

        with jax.named_scope("copyout%d" % r):
            pltpu.sync_copy(acc.at[pl.ds(rbase, RPT)],
                            outs[r].at[c, pl.ds(rbase, RPT)])
            if with_deg:
                pltpu.sync_copy(dacc.at[pl.ds(dbase, DPT)],
                                gouts[r].at[c, pl.ds(dbase, DPT)])
        if r + 1 < R:
            plsc.subcore_barrier()


def _sc_agg(z, edges, with_deg):
    out_type = [jax.ShapeDtypeStruct((NC, NPAD, D), jnp.float32)
                for _ in range(R)]
    if with_deg:
        out_type += [jax.ShapeDtypeStruct((NC, DPAD), jnp.float32)
                     for _ in range(R)]
    scratch = [
        pltpu.VMEM((ZROWS, D), jnp.float32),
        pltpu.VMEM((DPT,), jnp.float32),
        pltpu.VMEM((CH,), jnp.float32),
        pltpu.VMEM((CH,), jnp.int32),
        pltpu.VMEM((CH,), jnp.int32),
        pltpu.VMEM((CH,), jnp.int32),
        pltpu.VMEM((CH,), jnp.int32),
        pltpu.VMEM((CH, D), jnp.float32),
        pltpu.VMEM((CH, D), jnp.float32),
        pltpu.VMEM_SHARED((NPAD, D), jnp.float32),
        pltpu.VMEM_SHARED((DPAD,), jnp.float32),
        pltpu.SemaphoreType.DMA,
        pltpu.SemaphoreType.DMA,
    ]
    mesh = plsc.VectorSubcoreMesh(core_axis_name="c", subcore_axis_name="s",
                                  num_cores=NC, num_subcores=NS)
    fn = pl.kernel(
        functools.partial(_sc_agg_body, with_deg),
        out_type=tuple(out_type),
        mesh=mesh,
        scratch_types=tuple(scratch),
    )
    return fn(z, *edges)


def _tc_layer_body(leaky, a0, a1, a2, d00, d01, d10, d11, d20, d21,
                   w0, w1, w2, b0, b1, b2, out):
    aggs = (a0, a1, a2)
    degs = ((d00, d01), (d10, d11), (d20, d21))
    ws = (w0, w1, w2)
    acc = b0[...] + b1[...] + b2[...]
    for r in range(R):
        agg = aggs[r][0] + aggs[r][1]
        deg = degs[r][0][...] + degs[r][1][...]
        inv = 1.0 / jnp.maximum(deg, 1.0)
        acc = acc + jnp.dot(agg * inv, ws[r][...],
                            preferred_element_type=jnp.float32)
    if leaky:
        acc = jnp.where(acc > 0, acc, 0.01 * acc)
    out[...] = acc


BN = 400


def _tc_layer(aggs, degs, ws, bs, leaky):
    grid = (N // BN,)
    agg_spec = pl.BlockSpec((NC, BN, D), lambda i: (i * 0, i, i * 0))
    deg_spec = pl.BlockSpec((BN, 1), lambda i: (i, i * 0))
    w_spec = pl.BlockSpec((D, D), lambda i: (i * 0, i * 0))
    b_spec = pl.BlockSpec((1, D), lambda i: (i * 0, i * 0))
    in_specs = ([agg_spec] * R + [deg_spec] * (2 * R) + [w_spec] * R
                + [b_spec] * R)
    out = pl.pallas_call(
        functools.partial(_tc_layer_body, leaky),
        grid=grid,
        in_specs=in_specs,
        out_specs=pl.BlockSpec((BN, D), lambda i: (i, i * 0)),
        out_shape=jax.ShapeDtypeStruct((N, D), jnp.float32),
    )(*aggs, *degs, *ws, *bs)
    return out


def _prep_edges(ei):
    src = ei[0].astype(jnp.int32)
    dst = ei[1].astype(jnp.int32)
    src = jnp.pad(src, (0, EPAD - E))
    dummy = N + (jnp.arange(EPAD - E, dtype=jnp.int32) % (NPAD - N))
    dst = jnp.concatenate([dst, dummy])
    return src, dst


def kernel(x, edge_index_rel0, edge_index_rel1, edge_index_rel2,
           W1_0, b1_0, W1_1, b1_1, W1_2, b1_2,
           W2_0, b2_0, W2_1, b2_1, W2_2, b2_2):
    edges = []
    for ei in (edge_index_rel0, edge_index_rel1, edge_index_rel2):
        edges.extend(_prep_edges(ei))
    edges = tuple(edges)

    x = x.astype(jnp.float32)

    res1 = _sc_agg(x, edges, with_deg=True)
    aggs1 = res1[:R]
    deg_parts = res1[R:]
    degs = []
    for g in deg_parts:
        degs.append(g[0].reshape(DPAD, 1))
        degs.append(g[1].reshape(DPAD, 1))

    w1 = (W1_0.astype(jnp.float32), W1_1.astype(jnp.float32),
          W1_2.astype(jnp.float32))
    b1 = (b1_0.reshape(1, D).astype(jnp.float32),
          b1_1.reshape(1, D).astype(jnp.float32),
          b1_2.reshape(1, D).astype(jnp.float32))
    h = _tc_layer(aggs1, degs, w1, b1, leaky=True)

    aggs2 = _sc_agg(h, edges, with_deg=False)
    w2 = (W2_0.astype(jnp.float32), W2_1.astype(jnp.float32),
          W2_2.astype(jnp.float32))
    b2 = (b2_0.reshape(1, D).astype(jnp.float32),
          b2_1.reshape(1, D).astype(jnp.float32),
          b2_2.reshape(1, D).astype(jnp.float32))
    out = _tc_layer(aggs2, degs, w2, b2, leaky=False)
    return out


# speedup vs baseline: 1.2539x; 1.2539x over previous
import functools

import jax
import jax.numpy as jnp
from jax import lax
from jax.experimental import pallas as pl
from jax.experimental.pallas import tpu as pltpu
from jax.experimental.pallas import tpu_sc as plsc

N = 10000
D = 128
E = 100000
R = 3

NC = 2
NS = 16
NW = NC * NS

CH = 128
K0 = 40
K1 = 10
EPAD = NS * (K0 + K1) * CH

NPAD = 10112
RPT = NPAD // NS
ZROWS = RPT // 8
NZCOPY = RPT // ZROWS

DPAD = 10240
DPT = DPAD // NS


def _sc_agg_body(with_deg, z_ref, s0, d0, s1, d1, s2, d2, *rest):
    if with_deg:
        (o0, o1, o2, g0, g1, g2, zb, zb1, ones, sv0, sv1, dv0, dv1, rows0,
         rows1, acc, dacc, sem0, sem1) = rest
        gouts = (g0, g1, g2)
    else:
        (o0, o1, o2, zb, zb1, ones, sv0, sv1, dv0, dv1, rows0, rows1, acc,
         dacc, sem0, sem1) = rest
        gouts = (None, None, None)
    svs = (sv0, sv1)
    dvs = (dv0, dv1)
    rowss = (rows0, rows1)
    sems = (sem0, sem1)
    srcs = (s0, s1, s2)
    dsts = (d0, d1, d2)
    outs = (o0, o1, o2)

    c = lax.axis_index("c")
    s = lax.axis_index("s")
    w = c * NS + s

    def fill_zb(i, _):
        for k in range(D // 16):
            zb[i, pl.ds(k * 16, 16)] = jnp.zeros((16,), jnp.float32)
        return _

    lax.fori_loop(jnp.int32(0), jnp.int32(ZROWS), fill_zb, jnp.int32(0))

    def fill_zb1(i, _):
        zb1[pl.ds(i * 16, 16)] = jnp.zeros((16,), jnp.float32)
        return _

    lax.fori_loop(jnp.int32(0), jnp.int32(DPT // 16), fill_zb1, jnp.int32(0))

    if with_deg:
        def fill_ones(i, _):
            ones[pl.ds(i * 16, 16)] = jnp.ones((16,), jnp.float32)
            return _

        lax.fori_loop(jnp.int32(0), jnp.int32(CH // 16), fill_ones, jnp.int32(0))

    rbase = s * RPT
    dbase = s * DPT
    ebase = jnp.where(c == 0, s * (K0 * CH), NS * (K0 * CH) + s * (K1 * CH))
    kcount = jnp.where(c == 0, jnp.int32(K0), jnp.int32(K1))
    khalf = jnp.where(c == 0, jnp.int32(K0 // 2), jnp.int32(K1 // 2))

    for r in range(R):
        with jax.named_scope("zero%d" % r):
            for z in range(NZCOPY):
                pltpu.sync_copy(zb, acc.at[pl.ds(rbase + z * ZROWS, ZROWS)])
            if with_deg:
                pltpu.sync_copy(zb1, dacc.at[pl.ds(dbase, DPT)])
            plsc.subcore_barrier()

        def prefetch(i, p):
            off = ebase + i * CH
            pltpu.sync_copy(srcs[r].at[pl.ds(off, CH)], svs[p])
            pltpu.sync_copy(dsts[r].at[pl.ds(off, CH)], dvs[p])
            pltpu.async_copy(z_ref.at[svs[p]], rowss[p], sems[p])

        def consume(p):
            pltpu.make_async_copy(z_ref.at[svs[p]], rowss[p], sems[p]).wait()
            pltpu.sync_copy(rowss[p], acc.at[dvs[p]], add=True)
            if with_deg:
                pltpu.sync_copy(ones, dacc.at[dvs[p]], add=True)

        def pair(j, carry):
            consume(0)

            @pl.when(2 * j + 2 < kcount)
            def _():
                prefetch(2 * j + 2, 0)

            consume(1)

            @pl.when(2 * j + 3 < kcount)
            def _():
                prefetch(2 * j + 3, 1)

            return carry

        with jax.named_scope("chunks%d" % r):
            @pl.when(jnp.int32(0) < kcount)
            def _():
                prefetch(jnp.int32(0), 0)
                prefetch(jnp.int32(1), 1)

            lax.fori_loop(jnp.int32(0), khalf, pair, jnp.int32(0))
            plsc.subcore_barrier()

        with jax.named_scope("copyout%d" % r):
            pltpu.sync_copy(acc.at[pl.ds(rbase, RPT)],
                            outs[r].at[c, pl.ds(rbase, RPT)])
            if with_deg:
                pltpu.sync_copy(dacc.at[pl.ds(dbase, DPT)],
                                gouts[r].at[c, pl.ds(dbase, DPT)])
        if r + 1 < R:
            plsc.subcore_barrier()


def _sc_agg(z, edges, with_deg):
    out_type = [jax.ShapeDtypeStruct((NC, NPAD, D), jnp.float32)
                for _ in range(R)]
    if with_deg:
        out_type += [jax.ShapeDtypeStruct((NC, DPAD), jnp.float32)
                     for _ in range(R)]
    scratch = [
        pltpu.VMEM((ZROWS, D), jnp.float32),
        pltpu.VMEM((DPT,), jnp.float32),
        pltpu.VMEM((CH,), jnp.float32),
        pltpu.VMEM((CH,), jnp.int32),
        pltpu.VMEM((CH,), jnp.int32),
        pltpu.VMEM((CH,), jnp.int32),
        pltpu.VMEM((CH,), jnp.int32),
        pltpu.VMEM((CH, D), jnp.float32),
        pltpu.VMEM((CH, D), jnp.float32),
        pltpu.VMEM_SHARED((NPAD, D), jnp.float32),
        pltpu.VMEM_SHARED((DPAD,), jnp.float32),
        pltpu.SemaphoreType.DMA,
        pltpu.SemaphoreType.DMA,
    ]
    mesh = plsc.VectorSubcoreMesh(core_axis_name="c", subcore_axis_name="s",
                                  num_cores=NC, num_subcores=NS)
    fn = pl.kernel(
        functools.partial(_sc_agg_body, with_deg),
        out_type=tuple(out_type),
        mesh=mesh,
        scratch_types=tuple(scratch),
    )
    return fn(z, *edges)


def _tc_layer_body(leaky, a0, a1, a2, d00, d01, d10, d11, d20, d21,
                   w0, w1, w2, b0, b1, b2, out):
    aggs = (a0, a1, a2)
    degs = ((d00, d01), (d10, d11), (d20, d21))
    ws = (w0, w1, w2)
    acc = b0[...] + b1[...] + b2[...]
    for r in range(R):
        agg = aggs[r][0] + aggs[r][1]
        deg = degs[r][0][...] + degs[r][1][...]
        inv = 1.0 / jnp.maximum(deg, 1.0)
        acc = acc + jnp.dot(agg * inv, ws[r][...],
                            preferred_element_type=jnp.float32)
    if leaky:
        acc = jnp.where(acc > 0, acc, 0.01 * acc)
    out[...] = acc


BN = 400


def _tc_layer(aggs, degs, ws, bs, leaky):
    grid = (N // BN,)
    agg_spec = pl.BlockSpec((NC, BN, D), lambda i: (i * 0, i, i * 0))
    deg_spec = pl.BlockSpec((BN, 1), lambda i: (i, i * 0))
    w_spec = pl.BlockSpec((D, D), lambda i: (i * 0, i * 0))
    b_spec = pl.BlockSpec((1, D), lambda i: (i * 0, i * 0))
    in_specs = ([agg_spec] * R + [deg_spec] * (2 * R) + [w_spec] * R
                + [b_spec] * R)
    out = pl.pallas_call(
        functools.partial(_tc_layer_body, leaky),
        grid=grid,
        in_specs=in_specs,
        out_specs=pl.BlockSpec((BN, D), lambda i: (i, i * 0)),
        out_shape=jax.ShapeDtypeStruct((N, D), jnp.float32),
    )(*aggs, *degs, *ws, *bs)
    return out


def _prep_edges(ei):
    src = ei[0].astype(jnp.int32)
    dst = ei[1].astype(jnp.int32)
    src = jnp.pad(src, (0, EPAD - E))
    dummy = N + (jnp.arange(EPAD - E, dtype=jnp.int32) % (NPAD - N))
    dst = jnp.concatenate([dst, dummy])
    return src, dst


def kernel(x, edge_index_rel0, edge_index_rel1, edge_index_rel2,
           W1_0, b1_0, W1_1, b1_1, W1_2, b1_2,
           W2_0, b2_0, W2_1, b2_1, W2_2, b2_2):
    edges = []
    for ei in (edge_index_rel0, edge_index_rel1, edge_index_rel2):
        edges.extend(_prep_edges(ei))
    edges = tuple(edges)

    x = x.astype(jnp.float32)

    res1 = _sc_agg(x, edges, with_deg=True)
    aggs1 = res1[:R]
    deg_parts = res1[R:]
    degs = []
    for g in deg_parts:
        degs.append(g[0].reshape(DPAD, 1))
        degs.append(g[1].reshape(DPAD, 1))

    w1 = (W1_0.astype(jnp.float32), W1_1.astype(jnp.float32),
          W1_2.astype(jnp.float32))
    b1 = (b1_0.reshape(1, D).astype(jnp.float32),
          b1_1.reshape(1, D).astype(jnp.float32),
          b1_2.reshape(1, D).astype(jnp.float32))
    h = _tc_layer(aggs1, degs, w1, b1, leaky=True)

    aggs2 = _sc_agg(h, edges, with_deg=False)
    w2 = (W2_0.astype(jnp.float32), W2_1.astype(jnp.float32),
          W2_2.astype(jnp.float32))
    b2 = (b2_0.reshape(1, D).astype(jnp.float32),
          b2_1.reshape(1, D).astype(jnp.float32),
          b2_2.reshape(1, D).astype(jnp.float32))
    out = _tc_layer(aggs2, degs, w2, b2, leaky=False)
    return out
